# baseline (speedup 1.0000x reference)
import functools

import jax
import jax.numpy as jnp
import numpy as np
from jax import lax
from jax.experimental import pallas as pl
from jax.experimental.pallas import tpu as pltpu
from jax.experimental.pallas import tpu_sc as plsc

BATCH = 4
MODEL_DIM = 1024
MAX_LEN = 2048

NC = 2
NS = 16
LANES = 16
NW = NC * NS

B_TOTAL = BATCH * MAX_LEN
B_PER_W = B_TOTAL // NW
CHUNK = 32
N_CHUNKS = B_PER_W // CHUNK


def _pos_encoding_np(d, L):
    i = np.arange(d, dtype=np.float64)
    pos = np.arange(L, dtype=np.float64).reshape(-1, 1)
    angle = pos / (10000.0 ** (2.0 * i / d))
    return np.where((np.arange(d) % 2) == 0, np.sin(angle), np.cos(angle)).astype(
        np.float32
    )


_POS = _pos_encoding_np(MODEL_DIM, MAX_LEN)

_VECS_PER_CHUNK = CHUNK * MODEL_DIM // LANES


def _sc_body(table_hbm, idx_hbm, pos_hbm, out_hbm, idx_v, rows_v, pos_v, sem):
    wid = lax.axis_index("s") * NC + lax.axis_index("c")
    base = wid * B_PER_W
    pos_base = (wid % (MAX_LEN // B_PER_W)) * B_PER_W

    pltpu.sync_copy(idx_hbm.at[pl.ds(base, B_PER_W)], idx_v)

    for k in range(N_CHUNKS):
        gather = pltpu.async_copy(
            table_hbm.at[idx_v.at[pl.ds(k * CHUNK, CHUNK)]], rows_v, sem
        )
        pltpu.sync_copy(pos_hbm.at[pl.ds(pos_base + k * CHUNK, CHUNK)], pos_v)
        gather.wait()

        def add_body(i, _):
            r = i // (MODEL_DIM // LANES)
            off = (i % (MODEL_DIM // LANES)) * LANES
            rows_v[r, pl.ds(off, LANES)] = (
                rows_v[r, pl.ds(off, LANES)] + pos_v[r, pl.ds(off, LANES)]
            )
            return 0

        lax.fori_loop(0, _VECS_PER_CHUNK, add_body, 0)

        pltpu.sync_copy(rows_v, out_hbm.at[pl.ds(base + k * CHUNK, CHUNK)])


@jax.jit
def _embed(idx, table, pos):
    mesh = plsc.VectorSubcoreMesh(
        core_axis_name="c", subcore_axis_name="s", num_cores=NC, num_subcores=NS
    )
    fn = pl.kernel(
        _sc_body,
        out_type=jax.ShapeDtypeStruct((B_TOTAL, MODEL_DIM), jnp.float32),
        mesh=mesh,
        scratch_types=[
            pltpu.VMEM((B_PER_W,), jnp.int32),
            pltpu.VMEM((CHUNK, MODEL_DIM), jnp.float32),
            pltpu.VMEM((CHUNK, MODEL_DIM), jnp.float32),
            pltpu.SemaphoreType.DMA,
        ],
    )
    return fn(table, idx, pos)


def kernel(x, table):
    idx = x.reshape(-1).astype(jnp.int32)
    pos = jnp.asarray(_POS)
    out = _embed(idx, table, pos)
    return out.reshape(BATCH, MAX_LEN, MODEL_DIM)


# speedup vs baseline: 2.4679x; 2.4679x over previous
import jax
import jax.numpy as jnp
import numpy as np
from jax import lax
from jax.experimental import pallas as pl
from jax.experimental.pallas import tpu as pltpu
from jax.experimental.pallas import tpu_sc as plsc

BATCH = 4
MODEL_DIM = 1024
MAX_LEN = 2048

NC = 2
NS = 16
LANES = 16
NW = NC * NS

B_TOTAL = BATCH * MAX_LEN
T_PER_W = MAX_LEN // NW
CHUNK = 16
N_TC = T_PER_W // CHUNK
N_CHUNKS = N_TC * BATCH
NB = 5
GLEAD = 3

_VPC = CHUNK * MODEL_DIM // LANES
_VR = MODEL_DIM // LANES


def _pos_encoding_np(d, L):
    i = np.arange(d, dtype=np.float64)
    pos = np.arange(L, dtype=np.float64).reshape(-1, 1)
    angle = pos / (10000.0 ** (2.0 * i / d))
    return np.where((np.arange(d) % 2) == 0, np.sin(angle), np.cos(angle)).astype(
        np.float32
    )


_POS = _pos_encoding_np(MODEL_DIM, MAX_LEN)


def _sc_body(table_hbm, idx_hbm, pos_hbm, out_hbm, *scratch):
    idx_v = scratch[0]
    rows = scratch[1 : 1 + NB]
    pos_b = scratch[1 + NB : 3 + NB]
    gsem = scratch[3 + NB : 3 + 2 * NB]
    psem = scratch[3 + 2 * NB : 3 + 3 * NB]
    possem = scratch[3 + 3 * NB : 5 + 3 * NB]

    wid = lax.axis_index("s") * NC + lax.axis_index("c")
    t0 = wid * T_PER_W

    for b in range(BATCH):
        pltpu.sync_copy(
            idx_hbm.at[pl.ds(b * MAX_LEN + t0, T_PER_W)],
            idx_v.at[pl.ds(b * T_PER_W, T_PER_W)],
        )

    pos_desc = [None] * N_TC
    for tc in range(2):
        pos_desc[tc] = pltpu.async_copy(
            pos_hbm.at[pl.ds(t0 + tc * CHUNK, CHUNK)], pos_b[tc], possem[tc]
        )

    def issue_gather(n):
        tc, b = n // BATCH, n % BATCH
        return pltpu.async_copy(
            table_hbm.at[idx_v.at[pl.ds(b * T_PER_W + tc * CHUNK, CHUNK)]],
            rows[n % NB],
            gsem[n % NB],
        )

    gat = [None] * NB
    put = [None] * NB
    for n in range(GLEAD):
        gat[n % NB] = issue_gather(n)

    for c in range(N_CHUNKS):
        j = c % NB
        tc, b = c // BATCH, c % BATCH

        if c == BATCH:
            pos_desc[2] = pltpu.async_copy(
                pos_hbm.at[pl.ds(t0 + 2 * CHUNK, CHUNK)], pos_b[0], possem[0]
            )
        if c == 2 * BATCH:
            pos_desc[3] = pltpu.async_copy(
                pos_hbm.at[pl.ds(t0 + 3 * CHUNK, CHUNK)], pos_b[1], possem[1]
            )

        n = c + GLEAD
        if n < N_CHUNKS:
            if n >= NB:
                put[n % NB].wait()
            gat[n % NB] = issue_gather(n)

        gat[j].wait()
        if b == 0:
            pos_desc[tc].wait()

        rows_j = rows[j]
        pos_tc = pos_b[tc % 2]

        @plsc.parallel_loop(0, _VPC, unroll=8)
        def add_body(i):
            r = i >> 6
            off = pl.multiple_of((i & (_VR - 1)) << 4, LANES)
            rows_j[r, pl.ds(off, LANES)] = (
                rows_j[r, pl.ds(off, LANES)] + pos_tc[r, pl.ds(off, LANES)]
            )

        put[j] = pltpu.async_copy(
            rows_j,
            out_hbm.at[pl.ds(b * MAX_LEN + t0 + tc * CHUNK, CHUNK)],
            psem[j],
        )

    for c in range(N_CHUNKS - NB, N_CHUNKS):
        put[c % NB].wait()


@jax.jit
def _embed(idx, table, pos):
    mesh = plsc.VectorSubcoreMesh(
        core_axis_name="c", subcore_axis_name="s", num_cores=NC, num_subcores=NS
    )
    scratch = (
        [pltpu.VMEM((BATCH * T_PER_W,), jnp.int32)]
        + [pltpu.VMEM((CHUNK, MODEL_DIM), jnp.float32) for _ in range(NB)]
        + [pltpu.VMEM((CHUNK, MODEL_DIM), jnp.float32) for _ in range(2)]
        + [pltpu.SemaphoreType.DMA for _ in range(2 * NB + 2)]
    )
    fn = pl.kernel(
        _sc_body,
        out_type=jax.ShapeDtypeStruct((B_TOTAL, MODEL_DIM), jnp.float32),
        mesh=mesh,
        scratch_types=scratch,
    )
    return fn(table, idx, pos)


def kernel(x, table):
    idx = x.reshape(-1).astype(jnp.int32)
    pos = jnp.asarray(_POS)
    out = _embed(idx, table, pos)
    return out.reshape(BATCH, MAX_LEN, MODEL_DIM)


# speedup vs baseline: 2.5724x; 1.0423x over previous
import jax
import jax.numpy as jnp
import numpy as np
from jax import lax
from jax.experimental import pallas as pl
from jax.experimental.pallas import tpu as pltpu
from jax.experimental.pallas import tpu_sc as plsc

BATCH = 4
MODEL_DIM = 1024
MAX_LEN = 2048

NC = 2
NS = 16
LANES = 16
NW = NC * NS

B_TOTAL = BATCH * MAX_LEN
T_PER_W = MAX_LEN // NW
CHUNK = 16
N_TC = T_PER_W // CHUNK
N_CHUNKS = N_TC * BATCH
NB = 5
GLEAD = 3

_VPC = CHUNK * MODEL_DIM // LANES
_VR = MODEL_DIM // LANES


def _pos_encoding_np(d, L):
    i = np.arange(d, dtype=np.float64)
    pos = np.arange(L, dtype=np.float64).reshape(-1, 1)
    angle = pos / (10000.0 ** (2.0 * i / d))
    return np.where((np.arange(d) % 2) == 0, np.sin(angle), np.cos(angle)).astype(
        np.float32
    )


_POS = _pos_encoding_np(MODEL_DIM, MAX_LEN)


def _sc_body(table_hbm, idx_hbm, pos_hbm, out_hbm, *scratch):
    idx_v = scratch[0]
    rows = scratch[1 : 1 + NB]
    pos_b = scratch[1 + NB : 3 + NB]
    gsem = scratch[3 + NB : 3 + 2 * NB]
    psem = scratch[3 + 2 * NB : 3 + 3 * NB]
    possem = scratch[3 + 3 * NB : 5 + 3 * NB]

    wid = lax.axis_index("s") * NC + lax.axis_index("c")
    t0 = wid * T_PER_W

    for b in range(BATCH):
        pltpu.sync_copy(
            idx_hbm.at[pl.ds(b * MAX_LEN + t0, T_PER_W)],
            idx_v.at[pl.ds(b * T_PER_W, T_PER_W)],
        )

    pos_desc = [None] * N_TC
    for tc in range(2):
        pos_desc[tc] = pltpu.async_copy(
            pos_hbm.at[pl.ds((t0 + tc * CHUNK) * MODEL_DIM, CHUNK * MODEL_DIM)],
            pos_b[tc],
            possem[tc],
        )

    def issue_gather(n):
        tc, b = n // BATCH, n % BATCH
        return pltpu.async_copy(
            table_hbm.at[idx_v.at[pl.ds(b * T_PER_W + tc * CHUNK, CHUNK)]],
            rows[n % NB],
            gsem[n % NB],
        )

    gat = [None] * NB
    put = [None] * NB
    for n in range(GLEAD):
        gat[n % NB] = issue_gather(n)

    for c in range(N_CHUNKS):
        j = c % NB
        tc, b = c // BATCH, c % BATCH

        if c == BATCH:
            pos_desc[2] = pltpu.async_copy(
                pos_hbm.at[pl.ds((t0 + 2 * CHUNK) * MODEL_DIM, CHUNK * MODEL_DIM)],
                pos_b[0],
                possem[0],
            )
        if c == 2 * BATCH:
            pos_desc[3] = pltpu.async_copy(
                pos_hbm.at[pl.ds((t0 + 3 * CHUNK) * MODEL_DIM, CHUNK * MODEL_DIM)],
                pos_b[1],
                possem[1],
            )

        n = c + GLEAD
        if n < N_CHUNKS:
            if n >= NB:
                put[n % NB].wait()
            gat[n % NB] = issue_gather(n)

        gat[j].wait()
        if b == 0:
            pos_desc[tc].wait()

        rows_j = rows[j]
        pos_tc = pos_b[tc % 2]

        @plsc.parallel_loop(0, _VPC, unroll=8)
        def add_body(i):
            r = i >> 6
            off = pl.multiple_of((i & (_VR - 1)) << 4, LANES)
            poff = pl.multiple_of(i << 4, LANES)
            rows_j[r, pl.ds(off, LANES)] = (
                rows_j[r, pl.ds(off, LANES)] + pos_tc[pl.ds(poff, LANES)]
            )

        put[j] = pltpu.async_copy(
            rows_j,
            out_hbm.at[pl.ds(b * MAX_LEN + t0 + tc * CHUNK, CHUNK)],
            psem[j],
        )

    for c in range(N_CHUNKS - NB, N_CHUNKS):
        put[c % NB].wait()


@jax.jit
def _embed(idx, table, pos):
    mesh = plsc.VectorSubcoreMesh(
        core_axis_name="c", subcore_axis_name="s", num_cores=NC, num_subcores=NS
    )
    scratch = (
        [pltpu.VMEM((BATCH * T_PER_W,), jnp.int32)]
        + [pltpu.VMEM((CHUNK, MODEL_DIM), jnp.float32) for _ in range(NB)]
        + [pltpu.VMEM((CHUNK * MODEL_DIM,), jnp.float32) for _ in range(2)]
        + [pltpu.SemaphoreType.DMA for _ in range(2 * NB + 2)]
    )
    fn = pl.kernel(
        _sc_body,
        out_type=jax.ShapeDtypeStruct((B_TOTAL, MODEL_DIM), jnp.float32),
        mesh=mesh,
        scratch_types=scratch,
    )
    return fn(table, idx, pos)


def kernel(x, table):
    idx = x.reshape(-1).astype(jnp.int32)
    pos = jnp.asarray(_POS.reshape(-1))
    out = _embed(idx, table, pos)
    return out.reshape(BATCH, MAX_LEN, MODEL_DIM)


# speedup vs baseline: 2.5773x; 1.0019x over previous
import jax
import jax.numpy as jnp
import numpy as np
from jax import lax
from jax.experimental import pallas as pl
from jax.experimental.pallas import tpu as pltpu
from jax.experimental.pallas import tpu_sc as plsc

BATCH = 4
MODEL_DIM = 1024
MAX_LEN = 2048

NC = 2
NS = 16
LANES = 16
NW = NC * NS

B_TOTAL = BATCH * MAX_LEN
T_PER_W = MAX_LEN // NW
CHUNK = 16
N_TC = T_PER_W // CHUNK
N_CHUNKS = N_TC * BATCH
NB = 5
GLEAD = 3

_VPC = CHUNK * MODEL_DIM // LANES
_VR = MODEL_DIM // LANES


def _pos_encoding_np(d, L):
    i = np.arange(d, dtype=np.float64)
    pos = np.arange(L, dtype=np.float64).reshape(-1, 1)
    angle = pos / (10000.0 ** (2.0 * i / d))
    return np.where((np.arange(d) % 2) == 0, np.sin(angle), np.cos(angle)).astype(
        np.float32
    )


_POS = _pos_encoding_np(MODEL_DIM, MAX_LEN)

try:
    _POS_DEV = jnp.asarray(_POS.reshape(-1))
except Exception:
    _POS_DEV = None


def _sc_body(table_hbm, idx_hbm, pos_hbm, out_hbm, *scratch):
    idx_v = scratch[0]
    rows = scratch[1 : 1 + NB]
    pos_b = scratch[1 + NB : 3 + NB]
    gsem = scratch[3 + NB : 3 + 2 * NB]
    psem = scratch[3 + 2 * NB : 3 + 3 * NB]
    possem = scratch[3 + 3 * NB : 5 + 3 * NB]

    wid = lax.axis_index("s") * NC + lax.axis_index("c")
    t0 = wid * T_PER_W

    for b in range(BATCH):
        pltpu.sync_copy(
            idx_hbm.at[pl.ds(b * MAX_LEN + t0, T_PER_W)],
            idx_v.at[pl.ds(b * T_PER_W, T_PER_W)],
        )

    pos_desc = [None] * N_TC
    for tc in range(2):
        pos_desc[tc] = pltpu.async_copy(
            pos_hbm.at[pl.ds((t0 + tc * CHUNK) * MODEL_DIM, CHUNK * MODEL_DIM)],
            pos_b[tc],
            possem[tc],
        )

    def issue_gather(n):
        tc, b = n // BATCH, n % BATCH
        return pltpu.async_copy(
            table_hbm.at[idx_v.at[pl.ds(b * T_PER_W + tc * CHUNK, CHUNK)]],
            rows[n % NB],
            gsem[n % NB],
        )

    gat = [None] * NB
    put = [None] * NB
    for n in range(GLEAD):
        gat[n % NB] = issue_gather(n)

    for c in range(N_CHUNKS):
        j = c % NB
        tc, b = c // BATCH, c % BATCH

        if c == BATCH:
            pos_desc[2] = pltpu.async_copy(
                pos_hbm.at[pl.ds((t0 + 2 * CHUNK) * MODEL_DIM, CHUNK * MODEL_DIM)],
                pos_b[0],
                possem[0],
            )
        if c == 2 * BATCH:
            pos_desc[3] = pltpu.async_copy(
                pos_hbm.at[pl.ds((t0 + 3 * CHUNK) * MODEL_DIM, CHUNK * MODEL_DIM)],
                pos_b[1],
                possem[1],
            )

        n = c + GLEAD
        if n < N_CHUNKS:
            if n >= NB:
                put[n % NB].wait()
            gat[n % NB] = issue_gather(n)

        gat[j].wait()
        if b == 0:
            pos_desc[tc].wait()

        rows_j = rows[j]
        pos_tc = pos_b[tc % 2]

        @plsc.parallel_loop(0, _VPC, unroll=8)
        def add_body(i):
            r = i >> 6
            off = pl.multiple_of((i & (_VR - 1)) << 4, LANES)
            poff = pl.multiple_of(i << 4, LANES)
            rows_j[r, pl.ds(off, LANES)] = (
                rows_j[r, pl.ds(off, LANES)] + pos_tc[pl.ds(poff, LANES)]
            )

        put[j] = pltpu.async_copy(
            rows_j,
            out_hbm.at[pl.ds(b * MAX_LEN + t0 + tc * CHUNK, CHUNK)],
            psem[j],
        )

    for c in range(N_CHUNKS - NB, N_CHUNKS):
        put[c % NB].wait()


@jax.jit
def _embed(idx, table, pos):
    mesh = plsc.VectorSubcoreMesh(
        core_axis_name="c", subcore_axis_name="s", num_cores=NC, num_subcores=NS
    )
    scratch = (
        [pltpu.VMEM((BATCH * T_PER_W,), jnp.int32)]
        + [pltpu.VMEM((CHUNK, MODEL_DIM), jnp.float32) for _ in range(NB)]
        + [pltpu.VMEM((CHUNK * MODEL_DIM,), jnp.float32) for _ in range(2)]
        + [pltpu.SemaphoreType.DMA for _ in range(2 * NB + 2)]
    )
    fn = pl.kernel(
        _sc_body,
        out_type=jax.ShapeDtypeStruct((B_TOTAL, MODEL_DIM), jnp.float32),
        mesh=mesh,
        scratch_types=scratch,
    )
    return fn(table, idx, pos)


def kernel(x, table):
    idx = x.reshape(-1).astype(jnp.int32)
    pos = _POS_DEV if _POS_DEV is not None else jnp.asarray(_POS.reshape(-1))
    out = _embed(idx, table, pos)
    return out.reshape(BATCH, MAX_LEN, MODEL_DIM)
